# baseline (speedup 1.0000x reference)
import functools

import jax
import jax.numpy as jnp
from jax import lax
from jax.experimental import pallas as pl
from jax.experimental.pallas import tpu as pltpu
from jax.experimental.pallas import tpu_sc as plsc

ATOM_VOC = 128
CHIRAL_VOC = 4
BOND_VOC = 32
D_MODEL = 512
NHEAD = 16
B = 128
L = 128

_NEG_INF = float("-inf")

_NC = 2
_NS = 16
_LANES = 16
_NW = _NC * _NS
_ROWS = L * B
_RPW = _ROWS // _NW
_CH = 128
_NCH = _RPW // _CH


def _emb_sc_body(aT_hbm, cT_hbm, comb_hbm, out_hbm, a_v, c_v, idx_v, buf, sem):
    wid = lax.axis_index("s") * _NC + lax.axis_index("c")
    base = wid * _RPW
    pltpu.sync_copy(aT_hbm.at[pl.ds(base, _RPW)], a_v)
    pltpu.sync_copy(cT_hbm.at[pl.ds(base, _RPW)], c_v)
    for t in range(_RPW // _LANES):
        s = pl.ds(t * _LANES, _LANES)
        idx_v[s] = a_v[s] * CHIRAL_VOC + c_v[s]
    for j in range(_NCH):
        cp = pltpu.async_copy(
            comb_hbm.at[idx_v.at[pl.ds(j * _CH, _CH)]], buf, sem)
        cp.wait()
        pltpu.sync_copy(buf, out_hbm.at[pl.ds(base + j * _CH, _CH)])


_emb_sc = functools.partial(
    pl.kernel,
    out_type=jax.ShapeDtypeStruct((_ROWS, D_MODEL), jnp.float32),
    mesh=plsc.VectorSubcoreMesh(core_axis_name="c", subcore_axis_name="s"),
    scratch_types=[
        pltpu.VMEM((_RPW,), jnp.int32),
        pltpu.VMEM((_RPW,), jnp.int32),
        pltpu.VMEM((_RPW,), jnp.int32),
        pltpu.VMEM((_CH, D_MODEL), jnp.float32),
        pltpu.SemaphoreType.DMA,
    ],
)(_emb_sc_body)


def _tc_body(atoms2_ref, atomsT_ref, bonds_ref, a3p_ref, bwt_ref, out_ref):
    f32 = jnp.float32
    bf16 = jnp.bfloat16
    pid = pl.program_id(0)
    liota = lax.broadcasted_iota(jnp.int32, (L, ATOM_VOC), 1)

    arow_cat = atoms2_ref[0]
    viota2 = lax.broadcasted_iota(jnp.int32, (ATOM_VOC, 2 * L), 0)
    obt2 = (viota2 == arow_cat).astype(bf16)

    kb2 = lax.dot_general(a3p_ref[...], obt2, (((1,), (0,)), ((), ())),
                          preferred_element_type=f32)
    k3 = kb2.reshape(NHEAD, ATOM_VOC, 2 * L).astype(bf16)

    lane = lax.broadcasted_iota(jnp.int32, (NHEAD, ATOM_VOC), 1)
    lut = jnp.where(lane == 0, f32(0.0), bwt_ref[...])

    for k in range(2):
        k2 = jnp.concatenate(
            [k3[h][:, k * L:(k + 1) * L] for h in range(NHEAD)], axis=1)
        pidv = jnp.full((L, B), 2 * pid + k, dtype=jnp.int32)
        acol_b = jnp.take_along_axis(atomsT_ref[...], pidv, axis=1,
                                     mode="promise_in_bounds")
        obtT = (liota == acol_b).astype(bf16)
        ap_all = lax.dot_general(obtT, k2, (((1,), (0,)), ((), ())),
                                 preferred_element_type=f32)
        bonds2 = bonds_ref[k]
        maskj = arow_cat[:, k * L:(k + 1) * L] == 0
        for h in range(NHEAD):
            lut_h = jnp.broadcast_to(lut[h].reshape(1, ATOM_VOC), (L, L))
            bd = jnp.take_along_axis(lut_h, bonds2, axis=1,
                                     mode="promise_in_bounds")
            ap = ap_all[:, h * L:(h + 1) * L]
            out_ref[k, h] = jnp.where(maskj, _NEG_INF, ap + bd)


def kernel(atoms, chirals, bonds, atype_W, chiral_W, apair_W, bond_W):
    a3p = jnp.transpose(apair_W.reshape(ATOM_VOC, ATOM_VOC, NHEAD),
                        (2, 1, 0)).reshape(NHEAD * ATOM_VOC,
                                           ATOM_VOC).astype(jnp.bfloat16)
    bwt = jnp.pad(bond_W.T, ((0, 0), (0, ATOM_VOC - BOND_VOC)))
    comb = (atype_W.at[0].set(0.0)[:, None, :]
            + chiral_W.at[0].set(0.0)[None, :, :]
            ).reshape(ATOM_VOC * CHIRAL_VOC, D_MODEL)
    atoms2 = atoms.reshape(B // 2, 1, 2 * L)
    atomsT = atoms.T
    aT_flat = atomsT.reshape(_ROWS)
    cT_flat = chirals.T.reshape(_ROWS)

    emb_flat = _emb_sc(aT_flat, cT_flat, comb)
    emb = emb_flat.reshape(L, B, D_MODEL)

    apairs = pl.pallas_call(
        _tc_body,
        grid=(B // 2,),
        in_specs=[
            pl.BlockSpec((1, 1, 2 * L), lambda s: (s, 0, 0)),
            pl.BlockSpec((L, B), lambda s: (0, 0)),
            pl.BlockSpec((2, L, L), lambda s: (s, 0, 0)),
            pl.BlockSpec((NHEAD * ATOM_VOC, ATOM_VOC), lambda s: (0, 0)),
            pl.BlockSpec((NHEAD, ATOM_VOC), lambda s: (0, 0)),
        ],
        out_specs=pl.BlockSpec((2, NHEAD, L, L), lambda s: (s, 0, 0, 0)),
        out_shape=jax.ShapeDtypeStruct((B, NHEAD, L, L), jnp.float32),
        compiler_params=pltpu.CompilerParams(
            dimension_semantics=("arbitrary",),
        ),
    )(atoms2, atomsT, bonds, a3p, bwt)
    return emb, apairs
